# baseline (speedup 1.0000x reference)
import functools

import jax
import jax.numpy as jnp
from jax import lax
from jax.experimental import pallas as pl
from jax.experimental.pallas import tpu as pltpu
from jax.experimental.pallas import tpu_sc as plsc

NSAMPLE = 16
LEAKY = 0.1
ROWS = 256
MLP_ROWS = 512

SC_CORES = 2
SC_SUBCORES = 16
SC_WORKERS = SC_CORES * SC_SUBCORES
GATHER_CHUNK = 128


def _leaky(x):
    return jnp.where(x >= 0, x, LEAKY * x)


def _topk_body(q_ref, kT_ref, kf_ref, kxyz_ref, pos_wT_ref,
               idx_ref, tpre_ref):
    b = pl.program_id(0)
    q = q_ref[0]
    kT = kT_ref[0]
    f32 = jnp.float32

    R = q.shape[0]
    N = kT.shape[1]

    tpre_ref[0] = kf_ref[0] + jnp.dot(kxyz_ref[0], pos_wT_ref[...],
                                      preferred_element_type=f32)

    qk = jnp.dot(q, kT, preferred_element_type=f32)
    qn = jnp.sum(q * q, axis=1, keepdims=True)
    kn = jnp.sum(kT * kT, axis=0, keepdims=True)
    d = (-2.0 * qk + qn) + kn

    G = 4
    NG = N // G
    s = [d[:, p * NG:(p + 1) * NG] for p in range(G)]
    gi = lax.broadcasted_iota(jnp.int32, (R, NG), 1)
    ii = [gi + p * NG for p in range(G)]

    def ce(a, bb):
        sa, ia = a
        sb, ib = bb
        sw = sb < sa
        return ((jnp.where(sw, sb, sa), jnp.where(sw, ib, ia)),
                (jnp.where(sw, sa, sb), jnp.where(sw, ia, ib)))

    pv = [(s[p], ii[p]) for p in range(G)]
    for a, bb in ((0, 1), (2, 3), (1, 2), (0, 1), (2, 3), (1, 2)):
        pv[a], pv[bb] = ce(pv[a], pv[bb])
    (s0, i0), (s1, i1), (s2, i2), (s3, i3) = pv

    lane16 = lax.broadcasted_iota(jnp.int32, (R, NSAMPLE), 1)
    gbase = b * N
    BIG = jnp.int32(1 << 30)
    INF = jnp.float32(jnp.inf)

    def body(k, carry):
        s0, s1, s2, s3, i0, i1, i2, idxbuf = carry
        m = jnp.min(s0, axis=1, keepdims=True)
        eq = s0 == m
        gidx = jnp.min(jnp.where(eq, i0, BIG), axis=1, keepdims=True)
        oh = i0 == gidx
        s0 = jnp.where(oh, s1, s0)
        i0 = jnp.where(oh, i1, i0)
        s1 = jnp.where(oh, s2, s1)
        i1 = jnp.where(oh, i2, i1)
        s2 = jnp.where(oh, s3, s2)
        i2 = jnp.where(oh, i3, i2)
        s3 = jnp.where(oh, INF, s3)
        idxbuf = jnp.where(lane16 == k, gidx + gbase, idxbuf)
        return s0, s1, s2, s3, i0, i1, i2, idxbuf

    idxbuf0 = jnp.zeros((R, NSAMPLE), dtype=jnp.int32)
    out = lax.fori_loop(0, NSAMPLE, body,
                        (s0, s1, s2, s3, i0, i1, i2, idxbuf0))
    idx_ref[0] = out[-1]


def _topk_call(pcq, pck, featk):
    B, N, _ = pcq.shape
    D = featk.shape[-1]
    R = ROWS
    grid = (B, N // R)

    specs = [
        pl.BlockSpec((1, R, 3), lambda b, i: (b, i, 0)),
        pl.BlockSpec((1, 3, N), lambda b, i: (b, 0, 0)),
        pl.BlockSpec((1, R, D), lambda b, i: (b, i, 0)),
        pl.BlockSpec((1, R, 3), lambda b, i: (b, i, 0)),
        pl.BlockSpec((3, D), lambda b, i: (0, 0)),
    ]
    out_specs = [
        pl.BlockSpec((1, R, NSAMPLE), lambda b, i: (b, i, 0)),
        pl.BlockSpec((1, R, D), lambda b, i: (b, i, 0)),
    ]
    out_shape = [
        jax.ShapeDtypeStruct((B, N, NSAMPLE), jnp.int32),
        jax.ShapeDtypeStruct((B, N, D), jnp.float32),
    ]
    return pl.pallas_call(
        _topk_body, grid=grid, in_specs=specs,
        out_specs=out_specs, out_shape=out_shape,
    )


def _run_topk(pcq, pck, featk, pos_wT):
    kT = jnp.swapaxes(pck, 1, 2)
    return _topk_call(pcq, pck, featk)(pcq, kT, featk, pck, pos_wT)


def _sc_gather(idx_flat, table):
    T = idx_flat.shape[0]
    D = table.shape[1]
    per_w = T // SC_WORKERS
    ch = GATHER_CHUNK
    n_ch = per_w // ch
    mesh = plsc.VectorSubcoreMesh(core_axis_name="c", subcore_axis_name="s",
                                  num_cores=SC_CORES, num_subcores=SC_SUBCORES)

    @functools.partial(
        pl.kernel, mesh=mesh,
        compiler_params=pltpu.CompilerParams(use_tc_tiling_on_sc=False),
        out_type=jax.ShapeDtypeStruct((T, D), jnp.float32),
        scratch_types=[
            pltpu.VMEM((ch,), jnp.int32),
            pltpu.VMEM((ch, D), jnp.float32),
            pltpu.SemaphoreType.DMA,
        ],
    )
    def gk(idx_hbm, table_hbm, out_hbm, idx_v, rows_v, sem):
        wid = lax.axis_index("s") * SC_CORES + lax.axis_index("c")
        base = wid * per_w

        def body(c, carry):
            off = base + c * ch
            pltpu.sync_copy(idx_hbm.at[pl.ds(off, ch)], idx_v)
            pltpu.async_copy(table_hbm.at[idx_v], rows_v, sem).wait()
            pltpu.sync_copy(rows_v, out_hbm.at[pl.ds(off, ch)])
            return carry

        lax.fori_loop(0, n_ch, body, 0)

    return gk(idx_flat, table)


def _mlp_body(g_ref, qf_ref, q_ref, pos_wT_ref, pos_b_ref,
              w1T_ref, b1_ref, w2T_ref, b2_ref, tT_ref, tb_ref, out_ref):
    f32 = jnp.float32
    g = g_ref[0]
    qf = qf_ref[0]
    q = q_ref[0]
    R = qf.shape[0]
    D = qf.shape[1]

    base = qf + (pos_b_ref[...]
                 - jnp.dot(q, pos_wT_ref[...], preferred_element_type=f32))
    x = _leaky(g + base[:, None, :])
    x2 = x.reshape(R * NSAMPLE, D)
    x2 = _leaky(jnp.dot(x2, w1T_ref[...], preferred_element_type=f32)
                + b1_ref[...])
    x2 = _leaky(jnp.dot(x2, w2T_ref[...], preferred_element_type=f32)
                + b2_ref[...])
    x3 = x2.reshape(R, NSAMPLE, D)
    mx = x3[:, 0, :]
    for k in range(1, NSAMPLE):
        mx = jnp.maximum(mx, x3[:, k, :])
    out_ref[0] = jnp.dot(mx, tT_ref[...], preferred_element_type=f32) + tb_ref[...]


def _mlp_call(g, featq, pcq, pos_wT, pos_b2, w1T, b12, w2T, b22, tT, tb2):
    B, N, _, D = g.shape
    R = MLP_ROWS
    grid = (B, N // R)
    specs = [
        pl.BlockSpec((1, R, NSAMPLE, D), lambda b, i: (b, i, 0, 0)),
        pl.BlockSpec((1, R, D), lambda b, i: (b, i, 0)),
        pl.BlockSpec((1, R, 3), lambda b, i: (b, i, 0)),
        pl.BlockSpec((3, D), lambda b, i: (0, 0)),
        pl.BlockSpec((1, D), lambda b, i: (0, 0)),
        pl.BlockSpec((D, D), lambda b, i: (0, 0)),
        pl.BlockSpec((1, D), lambda b, i: (0, 0)),
        pl.BlockSpec((D, D), lambda b, i: (0, 0)),
        pl.BlockSpec((1, D), lambda b, i: (0, 0)),
        pl.BlockSpec((D, D), lambda b, i: (0, 0)),
        pl.BlockSpec((1, D), lambda b, i: (0, 0)),
    ]
    out_spec = pl.BlockSpec((1, R, D), lambda b, i: (b, i, 0))
    return pl.pallas_call(
        _mlp_body, grid=grid, in_specs=specs, out_specs=out_spec,
        out_shape=jax.ShapeDtypeStruct((B, N, D), jnp.float32),
    )(g, featq, pcq, pos_wT, pos_b2, w1T, b12, w2T, b22, tT, tb2)


def _cross_dir(pcq, pck, featq, featk, pos_wT, pos_b2, w1T, b12, w2T, b22,
               tT, tb2):
    B, N, _ = pcq.shape
    D = featq.shape[-1]
    idx, tpre = _run_topk(pcq, pck, featk, pos_wT)
    g_flat = _sc_gather(idx.reshape(-1), tpre.reshape(B * N, D))
    g = g_flat.reshape(B, N, NSAMPLE, D)
    return _mlp_call(g, featq, pcq, pos_wT, pos_b2, w1T, b12, w2T, b22,
                     tT, tb2)


@jax.jit
def kernel(pc1, pc2, feat1, feat2, pos_w, pos_b, mlp_w1, mlp_b1,
           mlp_w2, mlp_b2, t1_w, t1_b, t2_w, t2_b):
    pos_wT = pos_w.T
    pos_b2 = pos_b.reshape(1, -1)
    w1T = mlp_w1.T
    b12 = mlp_b1.reshape(1, -1)
    w2T = mlp_w2.T
    b22 = mlp_b2.reshape(1, -1)

    f1 = _cross_dir(pc1, pc2, feat1, feat2, pos_wT, pos_b2, w1T, b12,
                    w2T, b22, t1_w.T, t1_b.reshape(1, -1))
    f2 = _cross_dir(pc2, pc1, feat2, feat1, pos_wT, pos_b2, w1T, b12,
                    w2T, b22, t2_w.T, t2_b.reshape(1, -1))
    return (f1, f2)


# speedup vs baseline: 1.7389x; 1.7389x over previous
import functools

import jax
import jax.numpy as jnp
from jax import lax
from jax.experimental import pallas as pl
from jax.experimental.pallas import tpu as pltpu
from jax.experimental.pallas import tpu_sc as plsc

NSAMPLE = 16
LEAKY = 0.1
ROWS = 512
MLP_ROWS = 512

SC_CORES = 2
SC_SUBCORES = 16
SC_WORKERS = SC_CORES * SC_SUBCORES
GATHER_CHUNK = 128


def _leaky(x):
    return jnp.where(x >= 0, x, LEAKY * x)


def _topk_body(q_ref, kT_ref, kf_ref, kxyz_ref, pos_wT_ref,
               idx_ref, tpre_ref):
    b = pl.program_id(0)
    q = q_ref[0]
    kT = kT_ref[0]
    f32 = jnp.float32

    R = q.shape[0]
    N = kT.shape[1]

    tpre_ref[0] = kf_ref[0] + jnp.dot(kxyz_ref[0], pos_wT_ref[...],
                                      preferred_element_type=f32)

    qk = jnp.dot(q, kT, preferred_element_type=f32)
    qn = jnp.sum(q * q, axis=1, keepdims=True)
    kn = jnp.sum(kT * kT, axis=0, keepdims=True)
    d = (-2.0 * qk + qn) + kn

    iota = lax.broadcasted_iota(jnp.int32, (R, N), 1)
    gbase = b * N
    cols = []
    for _ in range(NSAMPLE):
        m = jnp.min(d, axis=1, keepdims=True)
        eq = d == m
        idx = jnp.min(jnp.where(eq, iota, N), axis=1, keepdims=True)
        d = jnp.where(iota == idx, jnp.float32(jnp.inf), d)
        cols.append(idx + gbase)
    idx_ref[0] = jnp.concatenate(cols, axis=1)


def _topk_call(pcq, pck, featk):
    B, N, _ = pcq.shape
    D = featk.shape[-1]
    R = ROWS
    grid = (B, N // R)

    specs = [
        pl.BlockSpec((1, R, 3), lambda b, i: (b, i, 0)),
        pl.BlockSpec((1, 3, N), lambda b, i: (b, 0, 0)),
        pl.BlockSpec((1, R, D), lambda b, i: (b, i, 0)),
        pl.BlockSpec((1, R, 3), lambda b, i: (b, i, 0)),
        pl.BlockSpec((3, D), lambda b, i: (0, 0)),
    ]
    out_specs = [
        pl.BlockSpec((1, R, NSAMPLE), lambda b, i: (b, i, 0)),
        pl.BlockSpec((1, R, D), lambda b, i: (b, i, 0)),
    ]
    out_shape = [
        jax.ShapeDtypeStruct((B, N, NSAMPLE), jnp.int32),
        jax.ShapeDtypeStruct((B, N, D), jnp.float32),
    ]
    return pl.pallas_call(
        _topk_body, grid=grid, in_specs=specs,
        out_specs=out_specs, out_shape=out_shape,
    )


def _run_topk(pcq, pck, featk, pos_wT):
    kT = jnp.swapaxes(pck, 1, 2)
    return _topk_call(pcq, pck, featk)(pcq, kT, featk, pck, pos_wT)


def _sc_gather(idx_flat, table):
    T = idx_flat.shape[0]
    D = table.shape[1]
    per_w = T // SC_WORKERS
    ch = GATHER_CHUNK
    n_ch = per_w // ch
    mesh = plsc.VectorSubcoreMesh(core_axis_name="c", subcore_axis_name="s",
                                  num_cores=SC_CORES, num_subcores=SC_SUBCORES)

    @functools.partial(
        pl.kernel, mesh=mesh,
        compiler_params=pltpu.CompilerParams(use_tc_tiling_on_sc=False),
        out_type=jax.ShapeDtypeStruct((T, D), jnp.float32),
        scratch_types=[
            pltpu.VMEM((ch,), jnp.int32),
            pltpu.VMEM((ch, D), jnp.float32),
            pltpu.SemaphoreType.DMA,
        ],
    )
    def gk(idx_hbm, table_hbm, out_hbm, idx_v, rows_v, sem):
        wid = lax.axis_index("s") * SC_CORES + lax.axis_index("c")
        base = wid * per_w

        def body(c, carry):
            off = base + c * ch
            pltpu.sync_copy(idx_hbm.at[pl.ds(off, ch)], idx_v)
            pltpu.async_copy(table_hbm.at[idx_v], rows_v, sem).wait()
            pltpu.sync_copy(rows_v, out_hbm.at[pl.ds(off, ch)])
            return carry

        lax.fori_loop(0, n_ch, body, 0)

    return gk(idx_flat, table)


def _mlp_body(g_ref, qf_ref, q_ref, pos_wT_ref, pos_b_ref,
              w1T_ref, b1_ref, w2T_ref, b2_ref, tT_ref, tb_ref, out_ref):
    f32 = jnp.float32
    g = g_ref[0]
    qf = qf_ref[0]
    q = q_ref[0]
    R = qf.shape[0]
    D = qf.shape[1]

    base = qf + (pos_b_ref[...]
                 - jnp.dot(q, pos_wT_ref[...], preferred_element_type=f32))
    x = _leaky(g + base[:, None, :])
    x2 = x.reshape(R * NSAMPLE, D)
    x2 = _leaky(jnp.dot(x2, w1T_ref[...], preferred_element_type=f32)
                + b1_ref[...])
    x2 = _leaky(jnp.dot(x2, w2T_ref[...], preferred_element_type=f32)
                + b2_ref[...])
    x3 = x2.reshape(R, NSAMPLE, D)
    mx = x3[:, 0, :]
    for k in range(1, NSAMPLE):
        mx = jnp.maximum(mx, x3[:, k, :])
    out_ref[0] = jnp.dot(mx, tT_ref[...], preferred_element_type=f32) + tb_ref[...]


def _mlp_call(g, featq, pcq, pos_wT, pos_b2, w1T, b12, w2T, b22, tT, tb2):
    B, N, _, D = g.shape
    R = MLP_ROWS
    grid = (B, N // R)
    specs = [
        pl.BlockSpec((1, R, NSAMPLE, D), lambda b, i: (b, i, 0, 0)),
        pl.BlockSpec((1, R, D), lambda b, i: (b, i, 0)),
        pl.BlockSpec((1, R, 3), lambda b, i: (b, i, 0)),
        pl.BlockSpec((3, D), lambda b, i: (0, 0)),
        pl.BlockSpec((1, D), lambda b, i: (0, 0)),
        pl.BlockSpec((D, D), lambda b, i: (0, 0)),
        pl.BlockSpec((1, D), lambda b, i: (0, 0)),
        pl.BlockSpec((D, D), lambda b, i: (0, 0)),
        pl.BlockSpec((1, D), lambda b, i: (0, 0)),
        pl.BlockSpec((D, D), lambda b, i: (0, 0)),
        pl.BlockSpec((1, D), lambda b, i: (0, 0)),
    ]
    out_spec = pl.BlockSpec((1, R, D), lambda b, i: (b, i, 0))
    return pl.pallas_call(
        _mlp_body, grid=grid, in_specs=specs, out_specs=out_spec,
        out_shape=jax.ShapeDtypeStruct((B, N, D), jnp.float32),
    )(g, featq, pcq, pos_wT, pos_b2, w1T, b12, w2T, b22, tT, tb2)


def _cross_dir(pcq, pck, featq, featk, pos_wT, pos_b2, w1T, b12, w2T, b22,
               tT, tb2):
    B, N, _ = pcq.shape
    D = featq.shape[-1]
    idx, tpre = _run_topk(pcq, pck, featk, pos_wT)
    g_flat = _sc_gather(idx.reshape(-1), tpre.reshape(B * N, D))
    g = g_flat.reshape(B, N, NSAMPLE, D)
    return _mlp_call(g, featq, pcq, pos_wT, pos_b2, w1T, b12, w2T, b22,
                     tT, tb2)


@jax.jit
def kernel(pc1, pc2, feat1, feat2, pos_w, pos_b, mlp_w1, mlp_b1,
           mlp_w2, mlp_b2, t1_w, t1_b, t2_w, t2_b):
    pos_wT = pos_w.T
    pos_b2 = pos_b.reshape(1, -1)
    w1T = mlp_w1.T
    b12 = mlp_b1.reshape(1, -1)
    w2T = mlp_w2.T
    b22 = mlp_b2.reshape(1, -1)

    f1 = _cross_dir(pc1, pc2, feat1, feat2, pos_wT, pos_b2, w1T, b12,
                    w2T, b22, t1_w.T, t1_b.reshape(1, -1))
    f2 = _cross_dir(pc2, pc1, feat2, feat1, pos_wT, pos_b2, w1T, b12,
                    w2T, b22, t2_w.T, t2_b.reshape(1, -1))
    return (f1, f2)


# speedup vs baseline: 1.8693x; 1.0750x over previous
import functools

import jax
import jax.numpy as jnp
from jax import lax
from jax.experimental import pallas as pl
from jax.experimental.pallas import tpu as pltpu
from jax.experimental.pallas import tpu_sc as plsc

NSAMPLE = 16
LEAKY = 0.1
ROWS = 512
MLP_ROWS = 512

SC_CORES = 2
SC_SUBCORES = 16
SC_WORKERS = SC_CORES * SC_SUBCORES
GATHER_CHUNK = 128


def _leaky(x):
    return jnp.where(x >= 0, x, LEAKY * x)


def _topk_body(q_ref, kT_ref, kf_ref, kxyz_ref, pos_wT_ref,
               idx_ref, tpre_ref):
    b = pl.program_id(0)
    q = q_ref[0]
    kT = kT_ref[0]
    f32 = jnp.float32

    R = q.shape[0]
    N = kT.shape[1]

    tpre_ref[0] = kf_ref[0] + jnp.dot(kxyz_ref[0], pos_wT_ref[...],
                                      preferred_element_type=f32)

    qk = jnp.dot(q, kT, preferred_element_type=f32)
    qn = jnp.sum(q * q, axis=1, keepdims=True)
    kn = jnp.sum(kT * kT, axis=0, keepdims=True)
    d = (-2.0 * qk + qn) + kn

    G = 4
    NG = N // G
    s = [d[:, p * NG:(p + 1) * NG] for p in range(G)]
    gi = lax.broadcasted_iota(jnp.int32, (R, NG), 1)
    ii = [gi + p * NG for p in range(G)]

    def ce(a, bb):
        sa, ia = a
        sb, ib = bb
        sw = sb < sa
        return ((jnp.where(sw, sb, sa), jnp.where(sw, ib, ia)),
                (jnp.where(sw, sa, sb), jnp.where(sw, ia, ib)))

    pv = [(s[p], ii[p]) for p in range(G)]
    for a, bb in ((0, 1), (2, 3), (1, 2), (0, 1), (2, 3), (1, 2)):
        pv[a], pv[bb] = ce(pv[a], pv[bb])
    (s0, i0), (s1, i1), (s2, i2), (s3, i3) = pv

    gbase = b * N
    BIG = jnp.int32(1 << 30)
    INF = jnp.float32(jnp.inf)
    cols = []
    for _ in range(NSAMPLE):
        m = jnp.min(s0, axis=1, keepdims=True)
        gidx = jnp.min(jnp.where(s0 == m, i0, BIG), axis=1, keepdims=True)
        oh = i0 == gidx
        s0 = jnp.where(oh, s1, s0)
        i0 = jnp.where(oh, i1, i0)
        s1 = jnp.where(oh, s2, s1)
        i1 = jnp.where(oh, i2, i1)
        s2 = jnp.where(oh, s3, s2)
        i2 = jnp.where(oh, i3, i2)
        s3 = jnp.where(oh, INF, s3)
        cols.append(gidx + gbase)
    idx_ref[0] = jnp.concatenate(cols, axis=1)


def _topk_call(pcq, pck, featk):
    B, N, _ = pcq.shape
    D = featk.shape[-1]
    R = ROWS
    grid = (B, N // R)

    specs = [
        pl.BlockSpec((1, R, 3), lambda b, i: (b, i, 0)),
        pl.BlockSpec((1, 3, N), lambda b, i: (b, 0, 0)),
        pl.BlockSpec((1, R, D), lambda b, i: (b, i, 0)),
        pl.BlockSpec((1, R, 3), lambda b, i: (b, i, 0)),
        pl.BlockSpec((3, D), lambda b, i: (0, 0)),
    ]
    out_specs = [
        pl.BlockSpec((1, R, NSAMPLE), lambda b, i: (b, i, 0)),
        pl.BlockSpec((1, R, D), lambda b, i: (b, i, 0)),
    ]
    out_shape = [
        jax.ShapeDtypeStruct((B, N, NSAMPLE), jnp.int32),
        jax.ShapeDtypeStruct((B, N, D), jnp.float32),
    ]
    return pl.pallas_call(
        _topk_body, grid=grid, in_specs=specs,
        out_specs=out_specs, out_shape=out_shape,
    )


def _run_topk(pcq, pck, featk, pos_wT):
    kT = jnp.swapaxes(pck, 1, 2)
    return _topk_call(pcq, pck, featk)(pcq, kT, featk, pck, pos_wT)


def _sc_gather(idx_flat, table):
    T = idx_flat.shape[0]
    D = table.shape[1]
    per_w = T // SC_WORKERS
    ch = GATHER_CHUNK
    n_ch = per_w // ch
    mesh = plsc.VectorSubcoreMesh(core_axis_name="c", subcore_axis_name="s",
                                  num_cores=SC_CORES, num_subcores=SC_SUBCORES)

    @functools.partial(
        pl.kernel, mesh=mesh,
        compiler_params=pltpu.CompilerParams(use_tc_tiling_on_sc=False),
        out_type=jax.ShapeDtypeStruct((T, D), jnp.float32),
        scratch_types=[
            pltpu.VMEM((ch,), jnp.int32),
            pltpu.VMEM((ch, D), jnp.float32),
            pltpu.SemaphoreType.DMA,
        ],
    )
    def gk(idx_hbm, table_hbm, out_hbm, idx_v, rows_v, sem):
        wid = lax.axis_index("s") * SC_CORES + lax.axis_index("c")
        base = wid * per_w

        def body(c, carry):
            off = base + c * ch
            pltpu.sync_copy(idx_hbm.at[pl.ds(off, ch)], idx_v)
            pltpu.async_copy(table_hbm.at[idx_v], rows_v, sem).wait()
            pltpu.sync_copy(rows_v, out_hbm.at[pl.ds(off, ch)])
            return carry

        lax.fori_loop(0, n_ch, body, 0)

    return gk(idx_flat, table)


def _mlp_body(g_ref, qf_ref, q_ref, pos_wT_ref, pos_b_ref,
              w1T_ref, b1_ref, w2T_ref, b2_ref, tT_ref, tb_ref, out_ref):
    f32 = jnp.float32
    g = g_ref[0]
    qf = qf_ref[0]
    q = q_ref[0]
    R = qf.shape[0]
    D = qf.shape[1]

    base = qf + (pos_b_ref[...]
                 - jnp.dot(q, pos_wT_ref[...], preferred_element_type=f32))
    x = _leaky(g + base[:, None, :])
    x2 = x.reshape(R * NSAMPLE, D)
    x2 = _leaky(jnp.dot(x2, w1T_ref[...], preferred_element_type=f32)
                + b1_ref[...])
    x2 = _leaky(jnp.dot(x2, w2T_ref[...], preferred_element_type=f32)
                + b2_ref[...])
    x3 = x2.reshape(R, NSAMPLE, D)
    mx = x3[:, 0, :]
    for k in range(1, NSAMPLE):
        mx = jnp.maximum(mx, x3[:, k, :])
    out_ref[0] = jnp.dot(mx, tT_ref[...], preferred_element_type=f32) + tb_ref[...]


def _mlp_call(g, featq, pcq, pos_wT, pos_b2, w1T, b12, w2T, b22, tT, tb2):
    B, N, _, D = g.shape
    R = MLP_ROWS
    grid = (B, N // R)
    specs = [
        pl.BlockSpec((1, R, NSAMPLE, D), lambda b, i: (b, i, 0, 0)),
        pl.BlockSpec((1, R, D), lambda b, i: (b, i, 0)),
        pl.BlockSpec((1, R, 3), lambda b, i: (b, i, 0)),
        pl.BlockSpec((3, D), lambda b, i: (0, 0)),
        pl.BlockSpec((1, D), lambda b, i: (0, 0)),
        pl.BlockSpec((D, D), lambda b, i: (0, 0)),
        pl.BlockSpec((1, D), lambda b, i: (0, 0)),
        pl.BlockSpec((D, D), lambda b, i: (0, 0)),
        pl.BlockSpec((1, D), lambda b, i: (0, 0)),
        pl.BlockSpec((D, D), lambda b, i: (0, 0)),
        pl.BlockSpec((1, D), lambda b, i: (0, 0)),
    ]
    out_spec = pl.BlockSpec((1, R, D), lambda b, i: (b, i, 0))
    return pl.pallas_call(
        _mlp_body, grid=grid, in_specs=specs, out_specs=out_spec,
        out_shape=jax.ShapeDtypeStruct((B, N, D), jnp.float32),
    )(g, featq, pcq, pos_wT, pos_b2, w1T, b12, w2T, b22, tT, tb2)


@jax.jit
def kernel(pc1, pc2, feat1, feat2, pos_w, pos_b, mlp_w1, mlp_b1,
           mlp_w2, mlp_b2, t1_w, t1_b, t2_w, t2_b):
    pos_wT = pos_w.T
    pos_b2 = pos_b.reshape(1, -1)
    w1T = mlp_w1.T
    b12 = mlp_b1.reshape(1, -1)
    w2T = mlp_w2.T
    b22 = mlp_b2.reshape(1, -1)
    B, N, _ = pc1.shape
    D = feat1.shape[-1]

    idx1, tpre1 = _run_topk(pc1, pc2, feat2, pos_wT)
    g1_flat = _sc_gather(idx1.reshape(-1), tpre1.reshape(B * N, D))
    idx2, tpre2 = _run_topk(pc2, pc1, feat1, pos_wT)
    g2_flat = _sc_gather(idx2.reshape(-1), tpre2.reshape(B * N, D))

    g1 = g1_flat.reshape(B, N, NSAMPLE, D)
    g2 = g2_flat.reshape(B, N, NSAMPLE, D)
    f1 = _mlp_call(g1, feat1, pc1, pos_wT, pos_b2, w1T, b12, w2T, b22,
                   t1_w.T, t1_b.reshape(1, -1))
    f2 = _mlp_call(g2, feat2, pc2, pos_wT, pos_b2, w1T, b12, w2T, b22,
                   t2_w.T, t2_b.reshape(1, -1))
    return (f1, f2)


# speedup vs baseline: 2.1434x; 1.1466x over previous
import functools

import jax
import jax.numpy as jnp
from jax import lax
from jax.experimental import pallas as pl
from jax.experimental.pallas import tpu as pltpu
from jax.experimental.pallas import tpu_sc as plsc

NSAMPLE = 16
LEAKY = 0.1
ROWS = 512
MLP_ROWS = 512

SC_CORES = 2
SC_SUBCORES = 16
SC_WORKERS = SC_CORES * SC_SUBCORES
GATHER_CHUNK = 128


def _leaky(x):
    return jnp.where(x >= 0, x, LEAKY * x)


def _topk_body(q_ref, kT_ref, kf_ref, kxyz_ref, pos_wT_ref,
               idx_ref, tpre_ref):
    b = pl.program_id(0)
    q = q_ref[0]
    kT = kT_ref[0]
    f32 = jnp.float32

    R = q.shape[0]
    N = kT.shape[1]

    tpre_ref[0] = kf_ref[0] + jnp.dot(kxyz_ref[0], pos_wT_ref[...],
                                      preferred_element_type=f32)

    qk = jnp.dot(q, kT, preferred_element_type=f32)
    qn = jnp.sum(q * q, axis=1, keepdims=True)
    kn = jnp.sum(kT * kT, axis=0, keepdims=True)
    d = (-2.0 * qk + qn) + kn

    G = 4
    NG = N // G
    s = [d[:, p * NG:(p + 1) * NG] for p in range(G)]
    gi = lax.broadcasted_iota(jnp.int32, (R, NG), 1)
    ii = [gi + p * NG for p in range(G)]

    def ce(a, bb):
        sa, ia = a
        sb, ib = bb
        sw = sb < sa
        return ((jnp.where(sw, sb, sa), jnp.where(sw, ib, ia)),
                (jnp.where(sw, sa, sb), jnp.where(sw, ia, ib)))

    pv = [(s[p], ii[p]) for p in range(G)]
    for a, bb in ((0, 1), (2, 3), (1, 2), (0, 1), (2, 3), (1, 2)):
        pv[a], pv[bb] = ce(pv[a], pv[bb])
    (s0, i0), (s1, i1), (s2, i2), (s3, i3) = pv

    gbase = b * N
    BIG = jnp.int32(1 << 30)
    INF = jnp.float32(jnp.inf)
    cols = []
    for _ in range(NSAMPLE):
        m = jnp.min(s0, axis=1, keepdims=True)
        gidx = jnp.min(jnp.where(s0 == m, i0, BIG), axis=1, keepdims=True)
        oh = i0 == gidx
        s0 = jnp.where(oh, s1, s0)
        i0 = jnp.where(oh, i1, i0)
        s1 = jnp.where(oh, s2, s1)
        i1 = jnp.where(oh, i2, i1)
        s2 = jnp.where(oh, s3, s2)
        i2 = jnp.where(oh, i3, i2)
        s3 = jnp.where(oh, INF, s3)
        cols.append(gidx + gbase)
    idx_ref[0] = jnp.concatenate(cols, axis=1)


def _topk_call(pcq, pck, featk):
    B, N, _ = pcq.shape
    D = featk.shape[-1]
    R = ROWS
    grid = (B, N // R)

    specs = [
        pl.BlockSpec((1, R, 3), lambda b, i: (b, i, 0)),
        pl.BlockSpec((1, 3, N), lambda b, i: (b, 0, 0)),
        pl.BlockSpec((1, R, D), lambda b, i: (b, i, 0)),
        pl.BlockSpec((1, R, 3), lambda b, i: (b, i, 0)),
        pl.BlockSpec((3, D), lambda b, i: (0, 0)),
    ]
    out_specs = [
        pl.BlockSpec((1, R, NSAMPLE), lambda b, i: (b, i, 0)),
        pl.BlockSpec((1, R, D), lambda b, i: (b, i, 0)),
    ]
    out_shape = [
        jax.ShapeDtypeStruct((B, N, NSAMPLE), jnp.int32),
        jax.ShapeDtypeStruct((B, N, D), jnp.float32),
    ]
    return pl.pallas_call(
        _topk_body, grid=grid, in_specs=specs,
        out_specs=out_specs, out_shape=out_shape,
    )


def _run_topk(pcq, pck, featk, pos_wT):
    kT = jnp.swapaxes(pck, 1, 2)
    return _topk_call(pcq, pck, featk)(pcq, kT, featk, pck, pos_wT)


def _sc_gather(idx_flat, table):
    T = idx_flat.shape[0]
    D = table.shape[1]
    per_w = T // SC_WORKERS
    ch = GATHER_CHUNK
    n_ch = per_w // ch
    mesh = plsc.VectorSubcoreMesh(core_axis_name="c", subcore_axis_name="s",
                                  num_cores=SC_CORES, num_subcores=SC_SUBCORES)

    @functools.partial(
        pl.kernel, mesh=mesh,
        compiler_params=pltpu.CompilerParams(use_tc_tiling_on_sc=False),
        out_type=jax.ShapeDtypeStruct((T, D), jnp.float32),
        scratch_types=[
            pltpu.VMEM((ch,), jnp.int32),
            pltpu.VMEM((ch, D), jnp.float32),
            pltpu.SemaphoreType.DMA,
        ],
    )
    def gk(idx_hbm, table_hbm, out_hbm, idx_v, rows_v, sem):
        wid = lax.axis_index("s") * SC_CORES + lax.axis_index("c")
        base = wid * per_w

        def body(c, carry):
            off = base + c * ch
            pltpu.sync_copy(idx_hbm.at[pl.ds(off, ch)], idx_v)
            pltpu.async_copy(table_hbm.at[idx_v], rows_v, sem).wait()
            pltpu.sync_copy(rows_v, out_hbm.at[pl.ds(off, ch)])
            return carry

        lax.fori_loop(0, n_ch, body, 0)

    return gk(idx_flat, table)


def _mlp_body(g_ref, qf_ref, q_ref, pos_wT_ref, pos_b_ref,
              w1_ref, b1_ref, w2_ref, b2_ref, tT_ref, tb_ref, out_ref):
    f32 = jnp.float32
    g = g_ref[0]
    qf = qf_ref[0]
    q = q_ref[0]
    R = qf.shape[0]
    D = qf.shape[1]

    base = qf + (pos_b_ref[...]
                 - jnp.dot(q, pos_wT_ref[...], preferred_element_type=f32))
    bl = jnp.concatenate([base, base, base, base], axis=1)
    basep = jnp.broadcast_to(bl[:, None, :], (R, 4, 128)).reshape(R * 4, 128)
    x = _leaky(g + basep)
    x = _leaky(jnp.dot(x, w1_ref[...], preferred_element_type=f32)
               + b1_ref[...])
    x = _leaky(jnp.dot(x, w2_ref[...], preferred_element_type=f32)
               + b2_ref[...])
    m = jnp.maximum(jnp.maximum(x[:, 0:D], x[:, D:2 * D]),
                    jnp.maximum(x[:, 2 * D:3 * D], x[:, 3 * D:]))
    m3 = m.reshape(R, 4, D)
    mx = jnp.maximum(jnp.maximum(m3[:, 0, :], m3[:, 1, :]),
                     jnp.maximum(m3[:, 2, :], m3[:, 3, :]))
    out_ref[0] = jnp.dot(mx, tT_ref[...], preferred_element_type=f32) + tb_ref[...]


def _mlp_call(gp, featq, pcq, pos_wT, pos_b2, w1p, b1p, w2p, b2p, tT, tb2):
    B, N, D = featq.shape
    R = MLP_ROWS
    RP = R * NSAMPLE * D // 128
    grid = (B, N // R)
    specs = [
        pl.BlockSpec((1, RP, 128), lambda b, i: (b, i, 0)),
        pl.BlockSpec((1, R, D), lambda b, i: (b, i, 0)),
        pl.BlockSpec((1, R, 3), lambda b, i: (b, i, 0)),
        pl.BlockSpec((3, D), lambda b, i: (0, 0)),
        pl.BlockSpec((1, D), lambda b, i: (0, 0)),
        pl.BlockSpec((128, 128), lambda b, i: (0, 0)),
        pl.BlockSpec((1, 128), lambda b, i: (0, 0)),
        pl.BlockSpec((128, 128), lambda b, i: (0, 0)),
        pl.BlockSpec((1, 128), lambda b, i: (0, 0)),
        pl.BlockSpec((D, D), lambda b, i: (0, 0)),
        pl.BlockSpec((1, D), lambda b, i: (0, 0)),
    ]
    out_spec = pl.BlockSpec((1, R, D), lambda b, i: (b, i, 0))
    return pl.pallas_call(
        _mlp_body, grid=grid, in_specs=specs, out_specs=out_spec,
        out_shape=jax.ShapeDtypeStruct((B, N, D), jnp.float32),
    )(gp, featq, pcq, pos_wT, pos_b2, w1p, b1p, w2p, b2p, tT, tb2)


@jax.jit
def kernel(pc1, pc2, feat1, feat2, pos_w, pos_b, mlp_w1, mlp_b1,
           mlp_w2, mlp_b2, t1_w, t1_b, t2_w, t2_b):
    pos_wT = pos_w.T
    pos_b2 = pos_b.reshape(1, -1)
    w1T = mlp_w1.T
    b12 = mlp_b1.reshape(1, -1)
    w2T = mlp_w2.T
    b22 = mlp_b2.reshape(1, -1)
    B, N, _ = pc1.shape
    D = feat1.shape[-1]

    eye4 = jnp.eye(4, dtype=jnp.float32)
    w1p = jnp.kron(eye4, w1T)
    w2p = jnp.kron(eye4, w2T)
    b1p = jnp.tile(mlp_b1, 4).reshape(1, 128)
    b2p = jnp.tile(mlp_b2, 4).reshape(1, 128)

    idx1, tpre1 = _run_topk(pc1, pc2, feat2, pos_wT)
    g1_flat = _sc_gather(idx1.reshape(-1), tpre1.reshape(B * N, D))
    idx2, tpre2 = _run_topk(pc2, pc1, feat1, pos_wT)
    g2_flat = _sc_gather(idx2.reshape(-1), tpre2.reshape(B * N, D))

    gp1 = g1_flat.reshape(B, N * NSAMPLE * D // 128, 128)
    gp2 = g2_flat.reshape(B, N * NSAMPLE * D // 128, 128)
    f1 = _mlp_call(gp1, feat1, pc1, pos_wT, pos_b2, w1p, b1p, w2p, b2p,
                   t1_w.T, t1_b.reshape(1, -1))
    f2 = _mlp_call(gp2, feat2, pc2, pos_wT, pos_b2, w1p, b1p, w2p, b2p,
                   t2_w.T, t2_b.reshape(1, -1))
    return (f1, f2)


# speedup vs baseline: 2.1679x; 1.0114x over previous
import functools

import jax
import jax.numpy as jnp
from jax import lax
from jax.experimental import pallas as pl
from jax.experimental.pallas import tpu as pltpu
from jax.experimental.pallas import tpu_sc as plsc

NSAMPLE = 16
LEAKY = 0.1
ROWS = 512
MLP_ROWS = 512

SC_CORES = 2
SC_SUBCORES = 16
SC_WORKERS = SC_CORES * SC_SUBCORES
GATHER_CHUNK = 128


def _leaky(x):
    return jnp.where(x >= 0, x, LEAKY * x)


def _topk_body(q_ref, kT_ref, kf_ref, kxyz_ref, pos_wT_ref,
               idx_ref, tpre_ref):
    b = pl.program_id(0)
    q = q_ref[0]
    kT = kT_ref[0]
    f32 = jnp.float32

    R = q.shape[0]
    N = kT.shape[1]

    tpre_ref[0] = kf_ref[0] + jnp.dot(kxyz_ref[0], pos_wT_ref[...],
                                      preferred_element_type=f32)

    qk = jnp.dot(q, kT, preferred_element_type=f32)
    qn = jnp.sum(q * q, axis=1, keepdims=True)
    kn = jnp.sum(kT * kT, axis=0, keepdims=True)
    d = (-2.0 * qk + qn) + kn

    G = 4
    NG = N // G
    s = [d[:, p * NG:(p + 1) * NG] for p in range(G)]
    gi = lax.broadcasted_iota(jnp.int32, (R, NG), 1)
    ii = [gi + p * NG for p in range(G)]

    def ce(a, bb):
        sa, ia = a
        sb, ib = bb
        sw = sb < sa
        return ((jnp.where(sw, sb, sa), jnp.where(sw, ib, ia)),
                (jnp.where(sw, sa, sb), jnp.where(sw, ia, ib)))

    pv = [(s[p], ii[p]) for p in range(G)]
    for a, bb in ((0, 1), (2, 3), (1, 2), (0, 1), (2, 3), (1, 2)):
        pv[a], pv[bb] = ce(pv[a], pv[bb])
    (s0, i0), (s1, i1), (s2, i2), (s3, i3) = pv

    gbase = b * N
    BIG = jnp.int32(1 << 30)
    INF = jnp.float32(jnp.inf)
    cols = []
    for _ in range(NSAMPLE):
        m = jnp.min(s0, axis=1, keepdims=True)
        gidx = jnp.min(jnp.where(s0 == m, i0, BIG), axis=1, keepdims=True)
        oh = i0 == gidx
        s0 = jnp.where(oh, s1, s0)
        i0 = jnp.where(oh, i1, i0)
        s1 = jnp.where(oh, s2, s1)
        i1 = jnp.where(oh, i2, i1)
        s2 = jnp.where(oh, s3, s2)
        i2 = jnp.where(oh, i3, i2)
        s3 = jnp.where(oh, INF, s3)
        cols.append(gidx + gbase)
    idx_ref[0] = jnp.concatenate(cols, axis=1)


def _topk_call(pcq, pck, featk):
    B, N, _ = pcq.shape
    D = featk.shape[-1]
    R = ROWS
    grid = (B, N // R)

    specs = [
        pl.BlockSpec((1, R, 3), lambda b, i: (b, i, 0)),
        pl.BlockSpec((1, 3, N), lambda b, i: (b, 0, 0)),
        pl.BlockSpec((1, R, D), lambda b, i: (b, i, 0)),
        pl.BlockSpec((1, R, 3), lambda b, i: (b, i, 0)),
        pl.BlockSpec((3, D), lambda b, i: (0, 0)),
    ]
    out_specs = [
        pl.BlockSpec((1, R, NSAMPLE), lambda b, i: (b, i, 0)),
        pl.BlockSpec((1, R, D), lambda b, i: (b, i, 0)),
    ]
    out_shape = [
        jax.ShapeDtypeStruct((B, N, NSAMPLE), jnp.int32),
        jax.ShapeDtypeStruct((B, N, D), jnp.float32),
    ]
    return pl.pallas_call(
        _topk_body, grid=grid, in_specs=specs,
        out_specs=out_specs, out_shape=out_shape,
    )


def _run_topk(pcq, pck, featk, pos_wT):
    kT = jnp.swapaxes(pck, 1, 2)
    return _topk_call(pcq, pck, featk)(pcq, kT, featk, pck, pos_wT)


def _sc_gather(idx_flat, table):
    T = idx_flat.shape[0]
    D = table.shape[1]
    per_w = T // SC_WORKERS
    ch = GATHER_CHUNK
    sup = 1024
    n_fire = sup // ch
    n_sup = per_w // sup
    mesh = plsc.VectorSubcoreMesh(core_axis_name="c", subcore_axis_name="s",
                                  num_cores=SC_CORES, num_subcores=SC_SUBCORES)

    @functools.partial(
        pl.kernel, mesh=mesh,
        compiler_params=pltpu.CompilerParams(use_tc_tiling_on_sc=False),
        out_type=jax.ShapeDtypeStruct((T, D), jnp.float32),
        scratch_types=[
            pltpu.VMEM((sup,), jnp.int32),
            pltpu.VMEM((sup, D), jnp.float32),
            pltpu.SemaphoreType.DMA,
        ],
    )
    def gk(idx_hbm, table_hbm, out_hbm, idx_v, rows_v, sem):
        wid = lax.axis_index("s") * SC_CORES + lax.axis_index("c")
        base = wid * per_w

        def body(c, carry):
            off = base + c * sup
            pltpu.sync_copy(idx_hbm.at[pl.ds(off, sup)], idx_v)
            cps = [
                pltpu.async_copy(
                    table_hbm.at[idx_v.at[pl.ds(j * ch, ch)]],
                    rows_v.at[pl.ds(j * ch, ch)], sem)
                for j in range(n_fire)
            ]
            for cp in cps:
                cp.wait()
            pltpu.sync_copy(rows_v, out_hbm.at[pl.ds(off, sup)])
            return carry

        lax.fori_loop(0, n_sup, body, 0)

    return gk(idx_flat, table)


def _mlp_body(g_ref, qf_ref, q_ref, pos_wT_ref, pos_b_ref,
              w1_ref, b1_ref, w2_ref, b2_ref, tT_ref, tb_ref, out_ref):
    f32 = jnp.float32
    g = g_ref[0]
    qf = qf_ref[0]
    q = q_ref[0]
    R = qf.shape[0]
    D = qf.shape[1]

    base = qf + (pos_b_ref[...]
                 - jnp.dot(q, pos_wT_ref[...], preferred_element_type=f32))
    bl = jnp.concatenate([base, base, base, base], axis=1)
    basep = jnp.broadcast_to(bl[:, None, :], (R, 4, 128)).reshape(R * 4, 128)
    x = _leaky(g + basep)
    x = _leaky(jnp.dot(x, w1_ref[...], preferred_element_type=f32)
               + b1_ref[...])
    x = _leaky(jnp.dot(x, w2_ref[...], preferred_element_type=f32)
               + b2_ref[...])
    m = jnp.maximum(jnp.maximum(x[:, 0:D], x[:, D:2 * D]),
                    jnp.maximum(x[:, 2 * D:3 * D], x[:, 3 * D:]))
    m3 = m.reshape(R, 4, D)
    mx = jnp.maximum(jnp.maximum(m3[:, 0, :], m3[:, 1, :]),
                     jnp.maximum(m3[:, 2, :], m3[:, 3, :]))
    out_ref[0] = jnp.dot(mx, tT_ref[...], preferred_element_type=f32) + tb_ref[...]


def _mlp_call(gp, featq, pcq, pos_wT, pos_b2, w1p, b1p, w2p, b2p, tT, tb2):
    B, N, D = featq.shape
    R = MLP_ROWS
    RP = R * NSAMPLE * D // 128
    grid = (B, N // R)
    specs = [
        pl.BlockSpec((1, RP, 128), lambda b, i: (b, i, 0)),
        pl.BlockSpec((1, R, D), lambda b, i: (b, i, 0)),
        pl.BlockSpec((1, R, 3), lambda b, i: (b, i, 0)),
        pl.BlockSpec((3, D), lambda b, i: (0, 0)),
        pl.BlockSpec((1, D), lambda b, i: (0, 0)),
        pl.BlockSpec((128, 128), lambda b, i: (0, 0)),
        pl.BlockSpec((1, 128), lambda b, i: (0, 0)),
        pl.BlockSpec((128, 128), lambda b, i: (0, 0)),
        pl.BlockSpec((1, 128), lambda b, i: (0, 0)),
        pl.BlockSpec((D, D), lambda b, i: (0, 0)),
        pl.BlockSpec((1, D), lambda b, i: (0, 0)),
    ]
    out_spec = pl.BlockSpec((1, R, D), lambda b, i: (b, i, 0))
    return pl.pallas_call(
        _mlp_body, grid=grid, in_specs=specs, out_specs=out_spec,
        out_shape=jax.ShapeDtypeStruct((B, N, D), jnp.float32),
    )(gp, featq, pcq, pos_wT, pos_b2, w1p, b1p, w2p, b2p, tT, tb2)


@jax.jit
def kernel(pc1, pc2, feat1, feat2, pos_w, pos_b, mlp_w1, mlp_b1,
           mlp_w2, mlp_b2, t1_w, t1_b, t2_w, t2_b):
    pos_wT = pos_w.T
    pos_b2 = pos_b.reshape(1, -1)
    w1T = mlp_w1.T
    b12 = mlp_b1.reshape(1, -1)
    w2T = mlp_w2.T
    b22 = mlp_b2.reshape(1, -1)
    B, N, _ = pc1.shape
    D = feat1.shape[-1]

    eye4 = jnp.eye(4, dtype=jnp.float32)
    w1p = jnp.kron(eye4, w1T)
    w2p = jnp.kron(eye4, w2T)
    b1p = jnp.tile(mlp_b1, 4).reshape(1, 128)
    b2p = jnp.tile(mlp_b2, 4).reshape(1, 128)

    idx1, tpre1 = _run_topk(pc1, pc2, feat2, pos_wT)
    g1_flat = _sc_gather(idx1.reshape(-1), tpre1.reshape(B * N, D))
    idx2, tpre2 = _run_topk(pc2, pc1, feat1, pos_wT)
    g2_flat = _sc_gather(idx2.reshape(-1), tpre2.reshape(B * N, D))

    gp1 = g1_flat.reshape(B, N * NSAMPLE * D // 128, 128)
    gp2 = g2_flat.reshape(B, N * NSAMPLE * D // 128, 128)
    f1 = _mlp_call(gp1, feat1, pc1, pos_wT, pos_b2, w1p, b1p, w2p, b2p,
                   t1_w.T, t1_b.reshape(1, -1))
    f2 = _mlp_call(gp2, feat2, pc2, pos_wT, pos_b2, w1p, b1p, w2p, b2p,
                   t2_w.T, t2_b.reshape(1, -1))
    return (f1, f2)
